# baseline (speedup 1.0000x reference)
import functools

import jax
import jax.numpy as jnp
from jax import lax
from jax.experimental import pallas as pl
from jax.experimental.pallas import tpu as pltpu
from jax.experimental.pallas import tpu_sc as plsc

NC, NS = 2, 16
NW = NC * NS
CHUNK = 512


@functools.partial(jax.jit, static_argnums=(2, 3))
def _gather(weight, idx, B, D):
    b_per_w = B // NW
    n_chunks = b_per_w // CHUNK
    mesh = plsc.VectorSubcoreMesh(
        core_axis_name="c", subcore_axis_name="s", num_cores=NC, num_subcores=NS
    )

    @functools.partial(
        pl.kernel,
        mesh=mesh,
        out_type=jax.ShapeDtypeStruct((B, D), jnp.float32),
        scratch_types=[
            pltpu.VMEM((CHUNK,), jnp.int32),
            pltpu.VMEM((CHUNK, D), jnp.float32),
            pltpu.SemaphoreType.DMA,
        ],
        compiler_params=pltpu.CompilerParams(use_tc_tiling_on_sc=False),
    )
    def k(table_hbm, idx_hbm, out_hbm, idx_v, rows_v, sem):
        wid = lax.axis_index("s") * NC + lax.axis_index("c")
        base = wid * b_per_w

        def body(i, carry):
            off = pl.multiple_of(base + i * CHUNK, CHUNK)
            pltpu.sync_copy(idx_hbm.at[pl.ds(off, CHUNK)], idx_v)
            pltpu.async_copy(table_hbm.at[idx_v], rows_v, sem).wait()
            pltpu.sync_copy(rows_v, out_hbm.at[pl.ds(off, CHUNK)])
            return carry

        lax.fori_loop(0, n_chunks, body, 0)

    return k(weight, idx)


def kernel(tokens, weight):
    S, T = tokens.shape
    V, D = weight.shape
    idx = tokens.reshape(S * T).astype(jnp.int32)
    out = _gather(weight, idx, S * T, D)
    return out.reshape(S, T, D)


# speedup vs baseline: 1.0424x; 1.0424x over previous
import functools

import jax
import jax.numpy as jnp
from jax import lax
from jax.experimental import pallas as pl
from jax.experimental.pallas import tpu as pltpu
from jax.experimental.pallas import tpu_sc as plsc

NC, NS = 2, 16
NW = NC * NS
CHUNK = 256
NBUF = 4


@functools.partial(jax.jit, static_argnums=(2, 3))
def _gather(weight, idx, B, D):
    b_per_w = B // NW
    n = b_per_w // CHUNK
    assert (n - NBUF) % NBUF == 0
    mesh = plsc.VectorSubcoreMesh(
        core_axis_name="c", subcore_axis_name="s", num_cores=NC, num_subcores=NS
    )

    @functools.partial(
        pl.kernel,
        mesh=mesh,
        out_type=jax.ShapeDtypeStruct((B, D), jnp.float32),
        scratch_types=[
            pltpu.VMEM((b_per_w,), jnp.int32),
            pltpu.VMEM((NBUF, CHUNK, D), jnp.float32),
        ]
        + [pltpu.SemaphoreType.DMA] * (2 * NBUF),
        compiler_params=pltpu.CompilerParams(use_tc_tiling_on_sc=False),
    )
    def k(table_hbm, idx_hbm, out_hbm, idx_all, rows, *sems):
        gsem, wsem = sems[:NBUF], sems[NBUF:]
        wid = lax.axis_index("s") * NC + lax.axis_index("c")
        base = pl.multiple_of(wid * b_per_w, b_per_w)
        pltpu.sync_copy(idx_hbm.at[pl.ds(base, b_per_w)], idx_all)

        def start_gather(t, b):
            pltpu.async_copy(
                table_hbm.at[idx_all.at[pl.ds(t * CHUNK, CHUNK)]],
                rows.at[b],
                gsem[b],
            )

        def finish_chunk(i, b):
            pltpu.make_async_copy(
                table_hbm.at[idx_all.at[pl.ds(0, CHUNK)]], rows.at[b], gsem[b]
            ).wait()
            pltpu.async_copy(
                rows.at[b],
                out_hbm.at[pl.ds(base + i * CHUNK, CHUNK)],
                wsem[b],
            )

        def wait_write(b):
            pltpu.make_async_copy(
                rows.at[b], out_hbm.at[pl.ds(base, CHUNK)], wsem[b]
            ).wait()

        for t in range(NBUF - 1):
            start_gather(t, t)
        finish_chunk(0, 0)
        start_gather(NBUF - 1, NBUF - 1)

        def body(g, carry):
            for jj in range(NBUF):
                i = 1 + g * NBUF + jj
                b = (1 + jj) % NBUF
                finish_chunk(i, b)
                wait_write(jj)
                start_gather(i + NBUF - 1, jj)
            return carry

        lax.fori_loop(0, (n - NBUF) // NBUF, body, 0)

        for jj in range(NBUF - 1):
            i = n - NBUF + 1 + jj
            finish_chunk(i, (1 + jj) % NBUF)
        for b in range(NBUF):
            wait_write(b)

    return k(weight, idx)


def kernel(tokens, weight):
    S, T = tokens.shape
    V, D = weight.shape
    idx = tokens.reshape(S * T).astype(jnp.int32)
    out = _gather(weight, idx, S * T, D)
    return out.reshape(S, T, D)
